# baseline (device time: 113105 ns/iter reference)
import jax
import jax.numpy as jnp
from jax import lax
from jax.experimental import pallas as pl
from jax.experimental.pallas import tpu as pltpu

N_Z = 4
S = 1024
D = 2048
DC = 128
H = 16
DH = 128
DR = 32
HB = H // N_Z
DHB = HB * DH
DRB = HB * DR
SCALE = (DH + DR) ** -0.5

DHH = DHB // 2

T_C, T_UK, T_UV, T_O0, T_O1 = 0, 1, 2, 3, 4
N_T = 5


def _body(x_ref, wdkv_ref, wuk_ref, wuv_ref, wq_hbm, wqr_hbm, wkr_ref,
          wo_ref, out_ref, gc, guk, guv, wqf, wqrf, qs, qrs, ks, vs,
          o_slots, send_sems, recv_sems, copy_sems):
    f32 = jnp.float32
    bf16 = jnp.bfloat16
    my_x = lax.axis_index("x")
    my_y = lax.axis_index("y")
    my_z = lax.axis_index("z")
    col0 = pl.multiple_of(my_z * DHB, DHB)
    qr0 = pl.multiple_of(my_z * DRB, DRB)

    wq_cp = pltpu.make_async_copy(
        wq_hbm.at[:, pl.ds(col0, DHB)], wqf, copy_sems.at[0])
    wq_cp.start()
    wqr_cp = pltpu.make_async_copy(
        wqr_hbm.at[:, pl.ds(qr0, DRB)], wqrf, copy_sems.at[1])
    wqr_cp.start()

    xv = x_ref[...]

    c_loc = jnp.dot(xv, wdkv_ref[...], preferred_element_type=f32)
    gc[my_z] = c_loc.astype(bf16)
    guk[my_z] = wuk_ref[:, pl.ds(col0, DHB)]
    guv[my_z] = wuv_ref[:, pl.ds(col0, DHB)]

    barrier = pltpu.get_barrier_semaphore()
    for dz in range(1, N_Z):
        pl.semaphore_signal(
            barrier, inc=1,
            device_id=(my_x, my_y, (my_z + dz) % N_Z),
            device_id_type=pl.DeviceIdType.MESH,
        )
    pl.semaphore_wait(barrier, N_Z - 1)

    def send(src_ref, dst_ref, t, j):
        rdma = pltpu.make_async_remote_copy(
            src_ref=src_ref,
            dst_ref=dst_ref,
            send_sem=send_sems.at[t, j],
            recv_sem=recv_sems.at[t, my_z],
            device_id=(my_x, my_y, j),
            device_id_type=pl.DeviceIdType.MESH,
        )
        rdma.start()
        return rdma

    rdmas = []
    for dz in range(1, N_Z):
        j = (my_z + dz) % N_Z
        jcol = pl.multiple_of(j * DHB, DHB)
        rdmas.append(send(gc.at[my_z], gc.at[my_z], T_C, j))
        rdmas.append(send(wuk_ref.at[:, pl.ds(jcol, DHB)], guk.at[my_z],
                          T_UK, j))
        rdmas.append(send(wuv_ref.at[:, pl.ds(jcol, DHB)], guv.at[my_z],
                          T_UV, j))

    wq_cp.wait()
    qs[...] = (jnp.dot(xv, wqf[...].astype(bf16),
                       preferred_element_type=f32) * SCALE).astype(bf16)
    wqr_cp.wait()
    qrs[...] = (jnp.dot(xv, wqrf[...].astype(bf16),
                        preferred_element_type=f32) * SCALE).astype(bf16)
    kr = jnp.dot(xv, wkr_ref[...], preferred_element_type=f32).astype(bf16)

    k_acc = jnp.dot(gc[my_z], guk[my_z], preferred_element_type=f32)
    v_acc = jnp.dot(gc[my_z], guv[my_z], preferred_element_type=f32)
    for dz in range(1, N_Z):
        j = (my_z + dz) % N_Z
        for t in (T_C, T_UK, T_UV):
            pltpu.make_async_remote_copy(
                src_ref=gc.at[my_z], dst_ref=(gc, guk, guv)[t].at[j],
                send_sem=send_sems.at[t, j], recv_sem=recv_sems.at[t, j],
                device_id=(my_x, my_y, j),
                device_id_type=pl.DeviceIdType.MESH,
            ).wait_recv()
        k_acc = k_acc + jnp.dot(gc[j], guk[j], preferred_element_type=f32)
        v_acc = v_acc + jnp.dot(gc[j], guv[j], preferred_element_type=f32)
    ks[...] = k_acc.astype(bf16)
    vs[...] = v_acc.astype(bf16)

    for i in range(HB):
        qh = qs[:, i * DH:(i + 1) * DH]
        qrh = qrs[:, i * DR:(i + 1) * DR]
        s = lax.dot_general(qh, ks[:, i * DH:(i + 1) * DH],
                            (((1,), (1,)), ((), ())),
                            preferred_element_type=f32)
        s = s + lax.dot_general(qrh, kr, (((1,), (1,)), ((), ())),
                                preferred_element_type=f32)
        e = jnp.exp(s)
        denom = jnp.sum(e, axis=-1, keepdims=True)
        o_un = jnp.dot(e.astype(bf16), vs[:, i * DH:(i + 1) * DH],
                       preferred_element_type=f32)
        half, sub = i // 2, i % 2
        o_slots[my_z, half, :, sub * DH:(sub + 1) * DH] = (
            (o_un / denom).astype(bf16))
        if sub == 1:
            t = T_O0 if half == 0 else T_O1
            for dz in range(1, N_Z):
                j = (my_z + dz) % N_Z
                rdmas.append(send(o_slots.at[my_z, half],
                                  o_slots.at[my_z, half], t, j))

    acc = None
    for half in range(2):
        h0 = pl.multiple_of(my_z * DHB + half * DHH, DHH)
        p = jnp.dot(o_slots[my_z, half], wo_ref[pl.ds(h0, DHH), :],
                    preferred_element_type=f32)
        acc = p if acc is None else acc + p
    out_ref[...] = acc
    for dz in range(1, N_Z):
        j = (my_z + dz) % N_Z
        for half, t in ((0, T_O0), (1, T_O1)):
            pltpu.make_async_remote_copy(
                src_ref=o_slots.at[my_z, half], dst_ref=o_slots.at[j, half],
                send_sem=send_sems.at[t, j], recv_sem=recv_sems.at[t, j],
                device_id=(my_x, my_y, j),
                device_id_type=pl.DeviceIdType.MESH,
            ).wait_recv()
            h0 = pl.multiple_of(j * DHB + half * DHH, DHH)
            out_ref[...] = out_ref[...] + jnp.dot(
                o_slots[j, half], wo_ref[pl.ds(h0, DHH), :],
                preferred_element_type=f32)

    for rdma in rdmas:
        rdma.wait_send()


def kernel(x, Wdkv, Wuk, Wuv, Wq, Wqr, Wkr, Wo):
    bf16 = jnp.bfloat16
    xb = x[0].astype(bf16)

    out = pl.pallas_call(
        _body,
        out_shape=jax.ShapeDtypeStruct((S, D), jnp.float32),
        in_specs=[
            pl.BlockSpec(memory_space=pltpu.VMEM),
            pl.BlockSpec(memory_space=pltpu.VMEM),
            pl.BlockSpec(memory_space=pltpu.VMEM),
            pl.BlockSpec(memory_space=pltpu.VMEM),
            pl.BlockSpec(memory_space=pl.ANY),
            pl.BlockSpec(memory_space=pl.ANY),
            pl.BlockSpec(memory_space=pltpu.VMEM),
            pl.BlockSpec(memory_space=pltpu.VMEM),
        ],
        out_specs=pl.BlockSpec(memory_space=pltpu.VMEM),
        scratch_shapes=[
            pltpu.VMEM((N_Z, S, DC), bf16),
            pltpu.VMEM((N_Z, DC, DHB), bf16),
            pltpu.VMEM((N_Z, DC, DHB), bf16),
            pltpu.VMEM((D, DHB), jnp.float32),
            pltpu.VMEM((D, DRB), jnp.float32),
            pltpu.VMEM((S, DHB), bf16),
            pltpu.VMEM((S, DRB), bf16),
            pltpu.VMEM((S, DHB), bf16),
            pltpu.VMEM((S, DHB), bf16),
            pltpu.VMEM((N_Z, 2, S, DHH), bf16),
            pltpu.SemaphoreType.DMA((N_T, N_Z)),
            pltpu.SemaphoreType.DMA((N_T, N_Z)),
            pltpu.SemaphoreType.DMA((2,)),
        ],
        compiler_params=pltpu.CompilerParams(
            collective_id=0, vmem_limit_bytes=128 * 1024 * 1024),
    )(xb, Wdkv.astype(bf16), Wuk.astype(bf16), Wuv.astype(bf16),
      Wq, Wqr, Wkr.astype(bf16), Wo.astype(bf16))
    return out.reshape(1, S, D)


# device time: 101015 ns/iter; 1.1197x vs baseline; 1.1197x over previous
import jax
import jax.numpy as jnp
from jax import lax
from jax.experimental import pallas as pl
from jax.experimental.pallas import tpu as pltpu

N_Z = 4
S = 1024
D = 2048
DC = 128
H = 16
DH = 128
DR = 32
HB = H // N_Z
DHB = HB * DH
DRB = HB * DR
SCALE = (DH + DR) ** -0.5

DHH = DHB // 2

T_C, T_UK, T_UV, T_O0, T_O1 = 0, 1, 2, 3, 4
N_T = 5


def _body(x_ref, wdkv_ref, wuk_ref, wuv_ref, wq_hbm, wqr_hbm, wkr_ref,
          wo_hbm, out_ref, gc, guk, guv, wqf, wqrf, wof, qs, qrs, ks, vs,
          o_slots, send_sems, recv_sems, copy_sems):
    f32 = jnp.float32
    bf16 = jnp.bfloat16
    my_x = lax.axis_index("x")
    my_y = lax.axis_index("y")
    my_z = lax.axis_index("z")
    col0 = pl.multiple_of(my_z * DHB, DHB)
    qr0 = pl.multiple_of(my_z * DRB, DRB)

    wq_cp = pltpu.make_async_copy(
        wq_hbm.at[:, pl.ds(col0, DHB)], wqf, copy_sems.at[0])
    wq_cp.start()
    wqr_cp = pltpu.make_async_copy(
        wqr_hbm.at[:, pl.ds(qr0, DRB)], wqrf, copy_sems.at[1])
    wqr_cp.start()

    xv = x_ref[...].astype(bf16)

    c_loc = jnp.dot(xv, wdkv_ref[...], preferred_element_type=f32)
    gc[my_z] = c_loc.astype(bf16)
    guk[my_z] = wuk_ref[:, pl.ds(col0, DHB)]
    guv[my_z] = wuv_ref[:, pl.ds(col0, DHB)]

    barrier = pltpu.get_barrier_semaphore()
    for dz in range(1, N_Z):
        pl.semaphore_signal(
            barrier, inc=1,
            device_id=(my_x, my_y, (my_z + dz) % N_Z),
            device_id_type=pl.DeviceIdType.MESH,
        )
    pl.semaphore_wait(barrier, N_Z - 1)

    def send(src_ref, dst_ref, t, j):
        rdma = pltpu.make_async_remote_copy(
            src_ref=src_ref,
            dst_ref=dst_ref,
            send_sem=send_sems.at[t, j],
            recv_sem=recv_sems.at[t, my_z],
            device_id=(my_x, my_y, j),
            device_id_type=pl.DeviceIdType.MESH,
        )
        rdma.start()
        return rdma

    rdmas = []
    for dz in range(1, N_Z):
        j = (my_z + dz) % N_Z
        jcol = pl.multiple_of(j * DHB, DHB)
        rdmas.append(send(gc.at[my_z], gc.at[my_z], T_C, j))
        rdmas.append(send(wuk_ref.at[:, pl.ds(jcol, DHB)], guk.at[my_z],
                          T_UK, j))
        rdmas.append(send(wuv_ref.at[:, pl.ds(jcol, DHB)], guv.at[my_z],
                          T_UV, j))

    wq_cp.wait()
    qs[...] = (jnp.dot(xv, wqf[...].astype(bf16),
                       preferred_element_type=f32) * SCALE).astype(bf16)
    wqr_cp.wait()
    qrs[...] = (jnp.dot(xv, wqrf[...].astype(bf16),
                        preferred_element_type=f32) * SCALE).astype(bf16)
    kr = jnp.dot(xv, wkr_ref[...], preferred_element_type=f32).astype(bf16)

    k_acc = jnp.dot(gc[my_z], guk[my_z], preferred_element_type=f32)
    v_acc = jnp.dot(gc[my_z], guv[my_z], preferred_element_type=f32)
    for dz in range(1, N_Z):
        j = (my_z + dz) % N_Z
        for t in (T_C, T_UK, T_UV):
            pltpu.make_async_remote_copy(
                src_ref=gc.at[my_z], dst_ref=(gc, guk, guv)[t].at[j],
                send_sem=send_sems.at[t, j], recv_sem=recv_sems.at[t, j],
                device_id=(my_x, my_y, j),
                device_id_type=pl.DeviceIdType.MESH,
            ).wait_recv()
        k_acc = k_acc + jnp.dot(gc[j], guk[j], preferred_element_type=f32)
        v_acc = v_acc + jnp.dot(gc[j], guv[j], preferred_element_type=f32)
    ks[...] = k_acc.astype(bf16)
    vs[...] = v_acc.astype(bf16)

    for i in range(HB):
        qh = qs[:, i * DH:(i + 1) * DH]
        qrh = qrs[:, i * DR:(i + 1) * DR]
        s = lax.dot_general(qh, ks[:, i * DH:(i + 1) * DH],
                            (((1,), (1,)), ((), ())),
                            preferred_element_type=f32)
        s = s + lax.dot_general(qrh, kr, (((1,), (1,)), ((), ())),
                                preferred_element_type=f32)
        e = jnp.exp(s)
        denom = jnp.sum(e, axis=-1, keepdims=True)
        o_un = jnp.dot(e.astype(bf16), vs[:, i * DH:(i + 1) * DH],
                       preferred_element_type=f32)
        half, sub = i // 2, i % 2
        o_slots[my_z, half, :, sub * DH:(sub + 1) * DH] = (
            (o_un / denom).astype(bf16))
        if sub == 1:
            t = T_O0 if half == 0 else T_O1
            for dz in range(1, N_Z):
                j = (my_z + dz) % N_Z
                rdmas.append(send(o_slots.at[my_z, half],
                                  o_slots.at[my_z, half], t, j))

    use_order = [(my_z, 0, None), (my_z, 1, None)]
    for dz in range(1, N_Z):
        j = (my_z + dz) % N_Z
        use_order.append((j, 0, T_O0))
        use_order.append((j, 1, T_O1))

    def wo_fetch(k):
        j, half, _ = use_order[k]
        h0 = pl.multiple_of(j * DHB + half * DHH, DHH)
        cp = pltpu.make_async_copy(
            wo_hbm.at[pl.ds(h0, DHH), :], wof.at[k % 2],
            copy_sems.at[2 + k % 2])
        cp.start()
        return cp

    fetches = [wo_fetch(0), wo_fetch(1)]
    for k, (j, half, t) in enumerate(use_order):
        if t is not None:
            pltpu.make_async_remote_copy(
                src_ref=o_slots.at[my_z, half], dst_ref=o_slots.at[j, half],
                send_sem=send_sems.at[t, j], recv_sem=recv_sems.at[t, j],
                device_id=(my_x, my_y, j),
                device_id_type=pl.DeviceIdType.MESH,
            ).wait_recv()
        fetches[k % 2].wait()
        p = jnp.dot(o_slots[j, half], wof[k % 2].astype(bf16),
                    preferred_element_type=f32)
        if k + 2 < len(use_order):
            fetches[k % 2] = wo_fetch(k + 2)
        if k == 0:
            out_ref[...] = p
        else:
            out_ref[...] = out_ref[...] + p

    for rdma in rdmas:
        rdma.wait_send()


def kernel(x, Wdkv, Wuk, Wuv, Wq, Wqr, Wkr, Wo):
    bf16 = jnp.bfloat16
    xb = x[0]

    out = pl.pallas_call(
        _body,
        out_shape=jax.ShapeDtypeStruct((S, D), jnp.float32),
        in_specs=[
            pl.BlockSpec(memory_space=pltpu.VMEM),
            pl.BlockSpec(memory_space=pltpu.VMEM),
            pl.BlockSpec(memory_space=pltpu.VMEM),
            pl.BlockSpec(memory_space=pltpu.VMEM),
            pl.BlockSpec(memory_space=pl.ANY),
            pl.BlockSpec(memory_space=pl.ANY),
            pl.BlockSpec(memory_space=pltpu.VMEM),
            pl.BlockSpec(memory_space=pl.ANY),
        ],
        out_specs=pl.BlockSpec(memory_space=pltpu.VMEM),
        scratch_shapes=[
            pltpu.VMEM((N_Z, S, DC), bf16),
            pltpu.VMEM((N_Z, DC, DHB), bf16),
            pltpu.VMEM((N_Z, DC, DHB), bf16),
            pltpu.VMEM((D, DHB), jnp.float32),
            pltpu.VMEM((D, DRB), jnp.float32),
            pltpu.VMEM((2, DHH, D), jnp.float32),
            pltpu.VMEM((S, DHB), bf16),
            pltpu.VMEM((S, DRB), bf16),
            pltpu.VMEM((S, DHB), bf16),
            pltpu.VMEM((S, DHB), bf16),
            pltpu.VMEM((N_Z, 2, S, DHH), bf16),
            pltpu.SemaphoreType.DMA((N_T, N_Z)),
            pltpu.SemaphoreType.DMA((N_T, N_Z)),
            pltpu.SemaphoreType.DMA((4,)),
        ],
        compiler_params=pltpu.CompilerParams(
            collective_id=0, vmem_limit_bytes=128 * 1024 * 1024),
    )(xb, Wdkv.astype(bf16), Wuk.astype(bf16), Wuv.astype(bf16),
      Wq, Wqr, Wkr.astype(bf16), Wo)
    return out.reshape(1, S, D)


# device time: 95268 ns/iter; 1.1872x vs baseline; 1.0603x over previous
import jax
import jax.numpy as jnp
from jax import lax
from jax.experimental import pallas as pl
from jax.experimental.pallas import tpu as pltpu

N_Z = 4
S = 1024
D = 2048
DC = 128
H = 16
DH = 128
DR = 32
HB = H // N_Z
DHB = HB * DH
DRB = HB * DR
SCALE = (DH + DR) ** -0.5

DHH = DHB // 2

T_C, T_UK, T_UV, T_O0, T_O1 = 0, 1, 2, 3, 4
N_T = 5


def _body(x_ref, wdkv_ref, wuk_ref, wuv_ref, wq_hbm, wqr_hbm, wkr_ref,
          wo_hbm, out_ref, gc, guk, guv, wukb, wuvb, wqf, wqrf, wof,
          qs, qrs, ks, vs, o_slots, send_sems, recv_sems, copy_sems):
    f32 = jnp.float32
    bf16 = jnp.bfloat16
    my_x = lax.axis_index("x")
    my_y = lax.axis_index("y")
    my_z = lax.axis_index("z")
    col0 = pl.multiple_of(my_z * DHB, DHB)
    qr0 = pl.multiple_of(my_z * DRB, DRB)

    wq_cp = pltpu.make_async_copy(
        wq_hbm.at[:, pl.ds(col0, DHB)], wqf, copy_sems.at[0])
    wq_cp.start()
    wqr_cp = pltpu.make_async_copy(
        wqr_hbm.at[:, pl.ds(qr0, DRB)], wqrf, copy_sems.at[1])
    wqr_cp.start()

    xv = x_ref[...].astype(bf16)
    wukb[...] = wuk_ref[...].astype(bf16)
    wuvb[...] = wuv_ref[...].astype(bf16)

    c_loc = jnp.dot(xv, wdkv_ref[...].astype(bf16),
                    preferred_element_type=f32)
    gc[my_z] = c_loc.astype(bf16)
    guk[my_z] = wukb[:, pl.ds(col0, DHB)]
    guv[my_z] = wuvb[:, pl.ds(col0, DHB)]

    barrier = pltpu.get_barrier_semaphore()
    for dz in range(1, N_Z):
        pl.semaphore_signal(
            barrier, inc=1,
            device_id=(my_x, my_y, (my_z + dz) % N_Z),
            device_id_type=pl.DeviceIdType.MESH,
        )
    pl.semaphore_wait(barrier, N_Z - 1)

    def send(src_ref, dst_ref, t, j):
        rdma = pltpu.make_async_remote_copy(
            src_ref=src_ref,
            dst_ref=dst_ref,
            send_sem=send_sems.at[t, j],
            recv_sem=recv_sems.at[t, my_z],
            device_id=(my_x, my_y, j),
            device_id_type=pl.DeviceIdType.MESH,
        )
        rdma.start()
        return rdma

    rdmas = []
    for dz in range(1, N_Z):
        j = (my_z + dz) % N_Z
        jcol = pl.multiple_of(j * DHB, DHB)
        rdmas.append(send(gc.at[my_z], gc.at[my_z], T_C, j))
        rdmas.append(send(wukb.at[:, pl.ds(jcol, DHB)], guk.at[my_z],
                          T_UK, j))
        rdmas.append(send(wuvb.at[:, pl.ds(jcol, DHB)], guv.at[my_z],
                          T_UV, j))

    wq_cp.wait()
    qs[...] = (jnp.dot(xv, wqf[...].astype(bf16),
                       preferred_element_type=f32) * SCALE).astype(bf16)
    wqr_cp.wait()
    qrs[...] = (jnp.dot(xv, wqrf[...].astype(bf16),
                        preferred_element_type=f32) * SCALE).astype(bf16)
    kr = jnp.dot(xv, wkr_ref[...].astype(bf16),
                 preferred_element_type=f32).astype(bf16)

    k_acc = jnp.dot(gc[my_z], guk[my_z], preferred_element_type=f32)
    v_acc = jnp.dot(gc[my_z], guv[my_z], preferred_element_type=f32)
    for dz in range(1, N_Z):
        j = (my_z + dz) % N_Z
        for t in (T_C, T_UK, T_UV):
            pltpu.make_async_remote_copy(
                src_ref=gc.at[my_z], dst_ref=(gc, guk, guv)[t].at[j],
                send_sem=send_sems.at[t, j], recv_sem=recv_sems.at[t, j],
                device_id=(my_x, my_y, j),
                device_id_type=pl.DeviceIdType.MESH,
            ).wait_recv()
        k_acc = k_acc + jnp.dot(gc[j], guk[j], preferred_element_type=f32)
        v_acc = v_acc + jnp.dot(gc[j], guv[j], preferred_element_type=f32)
    ks[...] = k_acc.astype(bf16)
    vs[...] = v_acc.astype(bf16)

    for i in range(HB):
        qh = qs[:, i * DH:(i + 1) * DH]
        qrh = qrs[:, i * DR:(i + 1) * DR]
        s = lax.dot_general(qh, ks[:, i * DH:(i + 1) * DH],
                            (((1,), (1,)), ((), ())),
                            preferred_element_type=f32)
        s = s + lax.dot_general(qrh, kr, (((1,), (1,)), ((), ())),
                                preferred_element_type=f32)
        e = jnp.exp(s)
        denom = jnp.sum(e, axis=-1, keepdims=True)
        o_un = jnp.dot(e.astype(bf16), vs[:, i * DH:(i + 1) * DH],
                       preferred_element_type=f32)
        half, sub = i // 2, i % 2
        o_slots[my_z, half, :, sub * DH:(sub + 1) * DH] = (
            (o_un / denom).astype(bf16))
        if sub == 1:
            t = T_O0 if half == 0 else T_O1
            for dz in range(1, N_Z):
                j = (my_z + dz) % N_Z
                rdmas.append(send(o_slots.at[my_z, half],
                                  o_slots.at[my_z, half], t, j))

    use_order = [(my_z, 0, None), (my_z, 1, None)]
    for dz in range(1, N_Z):
        j = (my_z + dz) % N_Z
        use_order.append((j, 0, T_O0))
        use_order.append((j, 1, T_O1))

    def wo_fetch(k):
        j, half, _ = use_order[k]
        h0 = pl.multiple_of(j * DHB + half * DHH, DHH)
        cp = pltpu.make_async_copy(
            wo_hbm.at[pl.ds(h0, DHH), :], wof.at[k % 2],
            copy_sems.at[2 + k % 2])
        cp.start()
        return cp

    fetches = [wo_fetch(0), wo_fetch(1)]
    for k, (j, half, t) in enumerate(use_order):
        if t is not None:
            pltpu.make_async_remote_copy(
                src_ref=o_slots.at[my_z, half], dst_ref=o_slots.at[j, half],
                send_sem=send_sems.at[t, j], recv_sem=recv_sems.at[t, j],
                device_id=(my_x, my_y, j),
                device_id_type=pl.DeviceIdType.MESH,
            ).wait_recv()
        fetches[k % 2].wait()
        p = jnp.dot(o_slots[j, half], wof[k % 2].astype(bf16),
                    preferred_element_type=f32)
        if k + 2 < len(use_order):
            fetches[k % 2] = wo_fetch(k + 2)
        if k == 0:
            out_ref[...] = p
        else:
            out_ref[...] = out_ref[...] + p

    for rdma in rdmas:
        rdma.wait_send()


def kernel(x, Wdkv, Wuk, Wuv, Wq, Wqr, Wkr, Wo):
    bf16 = jnp.bfloat16
    out = pl.pallas_call(
        _body,
        out_shape=jax.ShapeDtypeStruct((S, D), jnp.float32),
        in_specs=[
            pl.BlockSpec(memory_space=pltpu.VMEM),
            pl.BlockSpec(memory_space=pltpu.VMEM),
            pl.BlockSpec(memory_space=pltpu.VMEM),
            pl.BlockSpec(memory_space=pltpu.VMEM),
            pl.BlockSpec(memory_space=pl.ANY),
            pl.BlockSpec(memory_space=pl.ANY),
            pl.BlockSpec(memory_space=pltpu.VMEM),
            pl.BlockSpec(memory_space=pl.ANY),
        ],
        out_specs=pl.BlockSpec(memory_space=pltpu.VMEM),
        scratch_shapes=[
            pltpu.VMEM((N_Z, S, DC), bf16),
            pltpu.VMEM((N_Z, DC, DHB), bf16),
            pltpu.VMEM((N_Z, DC, DHB), bf16),
            pltpu.VMEM((DC, D), bf16),
            pltpu.VMEM((DC, D), bf16),
            pltpu.VMEM((D, DHB), jnp.float32),
            pltpu.VMEM((D, DRB), jnp.float32),
            pltpu.VMEM((2, DHH, D), jnp.float32),
            pltpu.VMEM((S, DHB), bf16),
            pltpu.VMEM((S, DRB), bf16),
            pltpu.VMEM((S, DHB), bf16),
            pltpu.VMEM((S, DHB), bf16),
            pltpu.VMEM((N_Z, 2, S, DHH), bf16),
            pltpu.SemaphoreType.DMA((N_T, N_Z)),
            pltpu.SemaphoreType.DMA((N_T, N_Z)),
            pltpu.SemaphoreType.DMA((4,)),
        ],
        compiler_params=pltpu.CompilerParams(
            collective_id=0, vmem_limit_bytes=128 * 1024 * 1024),
    )(x[0], Wdkv, Wuk, Wuv, Wq, Wqr, Wkr, Wo)
    return out.reshape(1, S, D)


# device time: 93421 ns/iter; 1.2107x vs baseline; 1.0198x over previous
import jax
import jax.numpy as jnp
from jax import lax
from jax.experimental import pallas as pl
from jax.experimental.pallas import tpu as pltpu

N_Z = 4
S = 1024
D = 2048
DC = 128
H = 16
DH = 128
DR = 32
HB = H // N_Z
DHB = HB * DH
DRB = HB * DR
SCALE = (DH + DR) ** -0.5

DHH = DHB // 2

T_C, T_UK, T_UV, T_O0, T_O1 = 0, 1, 2, 3, 4
N_T = 5


def _body(x_ref, wdkv_ref, wuk_ref, wuv_ref, wq_hbm, wqr_hbm, wkr_ref,
          wo_hbm, out_ref, gc, guk, guv, wukb, wuvb, wqf, wqrf, wof,
          qs, qrs, ks, vs, o_slots, send_sems, recv_sems, copy_sems):
    f32 = jnp.float32
    bf16 = jnp.bfloat16
    my_x = lax.axis_index("x")
    my_y = lax.axis_index("y")
    my_z = lax.axis_index("z")
    col0 = pl.multiple_of(my_z * DHB, DHB)
    qr0 = pl.multiple_of(my_z * DRB, DRB)

    wq_cp = pltpu.make_async_copy(
        wq_hbm.at[:, pl.ds(col0, DHB)], wqf, copy_sems.at[0])
    wq_cp.start()
    wqr_cp = pltpu.make_async_copy(
        wqr_hbm.at[:, pl.ds(qr0, DRB)], wqrf, copy_sems.at[1])
    wqr_cp.start()

    xv = x_ref[...].astype(bf16)
    wukb[...] = wuk_ref[...].astype(bf16)
    wuvb[...] = wuv_ref[...].astype(bf16)
    guk[my_z] = wukb[:, pl.ds(col0, DHB)]
    guv[my_z] = wuvb[:, pl.ds(col0, DHB)]

    barrier = pltpu.get_barrier_semaphore()
    for dz in range(1, N_Z):
        pl.semaphore_signal(
            barrier, inc=1,
            device_id=(my_x, my_y, (my_z + dz) % N_Z),
            device_id_type=pl.DeviceIdType.MESH,
        )
    pl.semaphore_wait(barrier, N_Z - 1)

    def send(src_ref, dst_ref, t, j):
        rdma = pltpu.make_async_remote_copy(
            src_ref=src_ref,
            dst_ref=dst_ref,
            send_sem=send_sems.at[t, j],
            recv_sem=recv_sems.at[t, my_z],
            device_id=(my_x, my_y, j),
            device_id_type=pl.DeviceIdType.MESH,
        )
        rdma.start()
        return rdma

    rdmas = []
    for dz in range(1, N_Z):
        j = (my_z + dz) % N_Z
        jcol = pl.multiple_of(j * DHB, DHB)
        rdmas.append(send(wukb.at[:, pl.ds(jcol, DHB)], guk.at[my_z],
                          T_UK, j))
        rdmas.append(send(wuvb.at[:, pl.ds(jcol, DHB)], guv.at[my_z],
                          T_UV, j))

    c_loc = jnp.dot(xv, wdkv_ref[...].astype(bf16),
                    preferred_element_type=f32)
    gc[my_z] = c_loc.astype(bf16)
    for dz in range(1, N_Z):
        j = (my_z + dz) % N_Z
        rdmas.append(send(gc.at[my_z], gc.at[my_z], T_C, j))

    wq_cp.wait()
    qs[...] = (jnp.dot(xv, wqf[...].astype(bf16),
                       preferred_element_type=f32) * SCALE).astype(bf16)
    wqr_cp.wait()
    qrs[...] = (jnp.dot(xv, wqrf[...].astype(bf16),
                        preferred_element_type=f32) * SCALE).astype(bf16)
    kr = jnp.dot(xv, wkr_ref[...].astype(bf16),
                 preferred_element_type=f32).astype(bf16)

    k_acc = jnp.dot(gc[my_z], guk[my_z], preferred_element_type=f32)
    v_acc = jnp.dot(gc[my_z], guv[my_z], preferred_element_type=f32)
    for dz in range(1, N_Z):
        j = (my_z + dz) % N_Z
        for t in (T_C, T_UK, T_UV):
            pltpu.make_async_remote_copy(
                src_ref=gc.at[my_z], dst_ref=(gc, guk, guv)[t].at[j],
                send_sem=send_sems.at[t, j], recv_sem=recv_sems.at[t, j],
                device_id=(my_x, my_y, j),
                device_id_type=pl.DeviceIdType.MESH,
            ).wait_recv()
        k_acc = k_acc + jnp.dot(gc[j], guk[j], preferred_element_type=f32)
        v_acc = v_acc + jnp.dot(gc[j], guv[j], preferred_element_type=f32)
    ks[...] = k_acc.astype(bf16)
    vs[...] = v_acc.astype(bf16)

    for i in range(HB):
        qh = qs[:, i * DH:(i + 1) * DH]
        qrh = qrs[:, i * DR:(i + 1) * DR]
        s = lax.dot_general(qh, ks[:, i * DH:(i + 1) * DH],
                            (((1,), (1,)), ((), ())),
                            preferred_element_type=f32)
        s = s + lax.dot_general(qrh, kr, (((1,), (1,)), ((), ())),
                                preferred_element_type=f32)
        e = jnp.exp(s)
        denom = jnp.sum(e, axis=-1, keepdims=True)
        o_un = jnp.dot(e.astype(bf16), vs[:, i * DH:(i + 1) * DH],
                       preferred_element_type=f32)
        half, sub = i // 2, i % 2
        o_slots[my_z, half, :, sub * DH:(sub + 1) * DH] = (
            (o_un / denom).astype(bf16))
        if sub == 1:
            t = T_O0 if half == 0 else T_O1
            for dz in range(1, N_Z):
                j = (my_z + dz) % N_Z
                rdmas.append(send(o_slots.at[my_z, half],
                                  o_slots.at[my_z, half], t, j))

    use_order = [(my_z, 0, None), (my_z, 1, None)]
    for dz in range(1, N_Z):
        j = (my_z + dz) % N_Z
        use_order.append((j, 0, T_O0))
        use_order.append((j, 1, T_O1))

    def wo_fetch(k):
        j, half, _ = use_order[k]
        h0 = pl.multiple_of(j * DHB + half * DHH, DHH)
        cp = pltpu.make_async_copy(
            wo_hbm.at[pl.ds(h0, DHH), :], wof.at[k % 2],
            copy_sems.at[2 + k % 2])
        cp.start()
        return cp

    fetches = [wo_fetch(0), wo_fetch(1)]
    for k, (j, half, t) in enumerate(use_order):
        if t is not None:
            pltpu.make_async_remote_copy(
                src_ref=o_slots.at[my_z, half], dst_ref=o_slots.at[j, half],
                send_sem=send_sems.at[t, j], recv_sem=recv_sems.at[t, j],
                device_id=(my_x, my_y, j),
                device_id_type=pl.DeviceIdType.MESH,
            ).wait_recv()
        fetches[k % 2].wait()
        p = jnp.dot(o_slots[j, half], wof[k % 2].astype(bf16),
                    preferred_element_type=f32)
        if k + 2 < len(use_order):
            fetches[k % 2] = wo_fetch(k + 2)
        if k == 0:
            out_ref[...] = p
        else:
            out_ref[...] = out_ref[...] + p

    for rdma in rdmas:
        rdma.wait_send()


def kernel(x, Wdkv, Wuk, Wuv, Wq, Wqr, Wkr, Wo):
    bf16 = jnp.bfloat16
    out = pl.pallas_call(
        _body,
        out_shape=jax.ShapeDtypeStruct((S, D), jnp.float32),
        in_specs=[
            pl.BlockSpec(memory_space=pltpu.VMEM),
            pl.BlockSpec(memory_space=pltpu.VMEM),
            pl.BlockSpec(memory_space=pltpu.VMEM),
            pl.BlockSpec(memory_space=pltpu.VMEM),
            pl.BlockSpec(memory_space=pl.ANY),
            pl.BlockSpec(memory_space=pl.ANY),
            pl.BlockSpec(memory_space=pltpu.VMEM),
            pl.BlockSpec(memory_space=pl.ANY),
        ],
        out_specs=pl.BlockSpec(memory_space=pltpu.VMEM),
        scratch_shapes=[
            pltpu.VMEM((N_Z, S, DC), bf16),
            pltpu.VMEM((N_Z, DC, DHB), bf16),
            pltpu.VMEM((N_Z, DC, DHB), bf16),
            pltpu.VMEM((DC, D), bf16),
            pltpu.VMEM((DC, D), bf16),
            pltpu.VMEM((D, DHB), jnp.float32),
            pltpu.VMEM((D, DRB), jnp.float32),
            pltpu.VMEM((2, DHH, D), jnp.float32),
            pltpu.VMEM((S, DHB), bf16),
            pltpu.VMEM((S, DRB), bf16),
            pltpu.VMEM((S, DHB), bf16),
            pltpu.VMEM((S, DHB), bf16),
            pltpu.VMEM((N_Z, 2, S, DHH), bf16),
            pltpu.SemaphoreType.DMA((N_T, N_Z)),
            pltpu.SemaphoreType.DMA((N_T, N_Z)),
            pltpu.SemaphoreType.DMA((4,)),
        ],
        compiler_params=pltpu.CompilerParams(
            collective_id=0, vmem_limit_bytes=128 * 1024 * 1024),
    )(x[0], Wdkv, Wuk, Wuv, Wq, Wqr, Wkr, Wo)
    return out.reshape(1, S, D)
